# Initial kernel scaffold; baseline (speedup 1.0000x reference)
import jax
import jax.numpy as jnp
from jax.experimental import pallas as pl


def kernel(x, qkv_w, dwx_w, dwx_b, dw_w, dw_b, pw_w, pw_b):
    raise NotImplementedError("write your pallas kernel here")


# speedup vs baseline: 4.8718x; 4.8718x over previous
import jax
import jax.numpy as jnp
from jax.experimental import pallas as pl
from jax.experimental.pallas import tpu as pltpu

DIM = 384
HEADS = 8
R = 4
TOPK = 4
HD = DIM // HEADS
SCALE = HD ** -0.5
H = 224
W = 224
HP = H // R
WP = W // R
N = HP * WP
NF = H * W
RR = R * R
BQ = 112
BM = 448
BW = 8
NEG = -1e30
HI = jax.lax.Precision.HIGHEST


def _matmul_kernel(x_ref, w_ref, o_ref):
    o_ref[...] = jnp.dot(x_ref[...], w_ref[...],
                         preferred_element_type=jnp.float32)


def _matmul(x, w, bm):
    m, k = x.shape
    n = w.shape[1]
    return pl.pallas_call(
        _matmul_kernel,
        grid=(m // bm,),
        in_specs=[pl.BlockSpec((bm, k), lambda i: (i, 0)),
                  pl.BlockSpec((k, n), lambda i: (0, 0))],
        out_specs=pl.BlockSpec((bm, n), lambda i: (i, 0)),
        out_shape=jax.ShapeDtypeStruct((m, n), jnp.float32),
    )(x, w)


def _matmul_bias_kernel(x_ref, w_ref, b_ref, o_ref):
    o_ref[...] = jnp.dot(x_ref[...], w_ref[...],
                         preferred_element_type=jnp.float32) + b_ref[...]


def _matmul_bias(x, w, b, bm):
    m, k = x.shape
    n = w.shape[1]
    return pl.pallas_call(
        _matmul_bias_kernel,
        grid=(m // bm,),
        in_specs=[pl.BlockSpec((bm, k), lambda i: (i, 0)),
                  pl.BlockSpec((k, n), lambda i: (0, 0)),
                  pl.BlockSpec((1, n), lambda i: (0, 0))],
        out_specs=pl.BlockSpec((bm, n), lambda i: (i, 0)),
        out_shape=jax.ShapeDtypeStruct((m, n), jnp.float32),
    )(x, w, b.reshape(1, n))


def _pool_kernel(x_ref, o_ref):
    v = x_ref[...]
    s = v.sum(axis=0)
    row = jax.lax.broadcasted_iota(jnp.int32, (WP, W), 0)
    col = jax.lax.broadcasted_iota(jnp.int32, (WP, W), 1) // R
    p = jnp.where(row == col, 1.0, 0.0).astype(jnp.float32)
    o_ref[...] = (jnp.dot(p, s, precision=HI, preferred_element_type=jnp.float32)
                  * (1.0 / (R * R)))[None]


def _pool(x_img):
    return pl.pallas_call(
        _pool_kernel,
        grid=(HP,),
        in_specs=[pl.BlockSpec((R, W, DIM), lambda i: (i, 0, 0))],
        out_specs=pl.BlockSpec((1, WP, DIM), lambda i: (i, 0, 0)),
        out_shape=jax.ShapeDtypeStruct((HP, WP, DIM), jnp.float32),
    )(x_img)


def _lowres_attn_kernel(q_ref, k_ref, v_ref, o_ref, idx_ref):
    q = q_ref[0]
    k = k_ref[0]
    v = v_ref[0]
    s = jax.lax.dot_general(q, k, (((1,), (1,)), ((), ())),
                            preferred_element_type=jnp.float32) * SCALE
    m = s.max(axis=1, keepdims=True)
    e = jnp.exp(s - m)
    d = e.sum(axis=1, keepdims=True)
    o_ref[0] = jax.lax.dot_general(e, v, (((1,), (0,)), ((), ())),
                                   preferred_element_type=jnp.float32) / d
    col = jax.lax.broadcasted_iota(jnp.int32, s.shape, 1)
    t = s
    for i in range(TOPK):
        mi = t.max(axis=1, keepdims=True)
        cand = jnp.where(t >= mi, col, jnp.int32(N))
        ji = cand.min(axis=1)
        idx_ref[0, :, i] = ji
        t = jnp.where(col == ji[:, None], NEG, t)


def _lowres_attn(q, k, v):
    return pl.pallas_call(
        _lowres_attn_kernel,
        grid=(HEADS, N // BQ),
        in_specs=[pl.BlockSpec((1, BQ, HD), lambda h, i: (h, i, 0)),
                  pl.BlockSpec((1, N, HD), lambda h, i: (h, 0, 0)),
                  pl.BlockSpec((1, N, HD), lambda h, i: (h, 0, 0))],
        out_specs=[pl.BlockSpec((1, BQ, HD), lambda h, i: (h, i, 0)),
                   pl.BlockSpec((1, BQ, TOPK), lambda h, i: (h, i, 0))],
        out_shape=[jax.ShapeDtypeStruct((HEADS, N, HD), jnp.float32),
                   jax.ShapeDtypeStruct((HEADS, N, TOPK), jnp.int32)],
    )(q, k, v)


def _win_attn_kernel(idx_ref, q_ref, k_ref, v_ref, o_ref):
    nb = pl.program_id(0)
    q = q_ref[...]
    kparts = []
    vparts = []
    for wloc in range(BW):
        n = nb * BW + wloc
        for t in range(TOPK):
            j = idx_ref[n * TOPK + t]
            kparts.append(k_ref[pl.ds(j * RR, RR), :])
            vparts.append(v_ref[pl.ds(j * RR, RR), :])
    kc = jnp.concatenate(kparts, axis=0)
    vc = jnp.concatenate(vparts, axis=0)
    s = jax.lax.dot_general(q, kc, (((1,), (1,)), ((), ())),
                            preferred_element_type=jnp.float32) * SCALE
    rw = jax.lax.broadcasted_iota(jnp.int32, s.shape, 0) // RR
    cw = jax.lax.broadcasted_iota(jnp.int32, s.shape, 1) // (TOPK * RR)
    s = jnp.where(rw == cw, s, NEG)
    m = s.max(axis=1, keepdims=True)
    e = jnp.exp(s - m)
    d = e.sum(axis=1, keepdims=True)
    o_ref[...] = jax.lax.dot_general(e, vc, (((1,), (0,)), ((), ())),
                                     preferred_element_type=jnp.float32) / d


def _win_attn_head(idx_flat, qf, kf, vf):
    return pl.pallas_call(
        _win_attn_kernel,
        grid_spec=pltpu.PrefetchScalarGridSpec(
            num_scalar_prefetch=1,
            grid=(N // BW,),
            in_specs=[pl.BlockSpec((BW * RR, HD), lambda i, s_: (i, 0)),
                      pl.BlockSpec((NF, HD), lambda i, s_: (0, 0)),
                      pl.BlockSpec((NF, HD), lambda i, s_: (0, 0))],
            out_specs=pl.BlockSpec((BW * RR, HD), lambda i, s_: (i, 0)),
        ),
        out_shape=jax.ShapeDtypeStruct((NF, HD), jnp.float32),
        compiler_params=pltpu.CompilerParams(vmem_limit_bytes=100 * 1024 * 1024),
    )(idx_flat, qf, kf, vf)


def _dwmix_kernel(x0_ref, x1_ref, x2_ref, y_ref, w_ref, b_ref, o_ref):
    acc = jnp.broadcast_to(b_ref[0][None, None, :], (1, W, DIM))
    xs = (x0_ref, x1_ref, x2_ref)
    for dr in range(3):
        xv = xs[dr][...]
        for dc in range(3):
            acc = acc + xv[:, dc:dc + W, :] * w_ref[3 * dr + dc][None, None, :]
    o_ref[...] = 0.5 * acc + 0.5 * y_ref[...]


def _dwmix(xpad, yw, w9, b):
    return pl.pallas_call(
        _dwmix_kernel,
        grid=(H,),
        in_specs=[pl.BlockSpec((1, W + 2, DIM), lambda i: (i, 0, 0)),
                  pl.BlockSpec((1, W + 2, DIM), lambda i: (i + 1, 0, 0)),
                  pl.BlockSpec((1, W + 2, DIM), lambda i: (i + 2, 0, 0)),
                  pl.BlockSpec((1, W, DIM), lambda i: (i, 0, 0)),
                  pl.BlockSpec((9, DIM), lambda i: (0, 0)),
                  pl.BlockSpec((1, DIM), lambda i: (0, 0))],
        out_specs=pl.BlockSpec((1, W, DIM), lambda i: (i, 0, 0)),
        out_shape=jax.ShapeDtypeStruct((H, W, DIM), jnp.float32),
    )(xpad, xpad, xpad, yw, w9, b.reshape(1, DIM))


def _dw_kernel(x0_ref, x1_ref, x2_ref, w_ref, b_ref, o_ref):
    acc = jnp.broadcast_to(b_ref[0][None, None, :], (1, W, DIM))
    xs = (x0_ref, x1_ref, x2_ref)
    for dr in range(3):
        xv = xs[dr][...]
        for dc in range(3):
            acc = acc + xv[:, dc:dc + W, :] * w_ref[3 * dr + dc][None, None, :]
    o_ref[...] = acc


def _dw(xpad, w9, b):
    return pl.pallas_call(
        _dw_kernel,
        grid=(H,),
        in_specs=[pl.BlockSpec((1, W + 2, DIM), lambda i: (i, 0, 0)),
                  pl.BlockSpec((1, W + 2, DIM), lambda i: (i + 1, 0, 0)),
                  pl.BlockSpec((1, W + 2, DIM), lambda i: (i + 2, 0, 0)),
                  pl.BlockSpec((9, DIM), lambda i: (0, 0)),
                  pl.BlockSpec((1, DIM), lambda i: (0, 0))],
        out_specs=pl.BlockSpec((1, W, DIM), lambda i: (i, 0, 0)),
        out_shape=jax.ShapeDtypeStruct((H, W, DIM), jnp.float32),
    )(xpad, xpad, xpad, w9, b.reshape(1, DIM))


def kernel(x, qkv_w, dwx_w, dwx_b, dw_w, dw_b, pw_w, pw_b):
    x_img = x.reshape(H, W, DIM)
    wt = qkv_w.T

    qkv_full = _matmul(x_img.reshape(NF, DIM), wt, bm=BM)

    xf = _pool(x_img).reshape(N, DIM)
    qkv_d = _matmul(xf, wt, bm=BM)
    qkv_d = qkv_d.reshape(N, 3, HEADS, HD).transpose(1, 2, 0, 3)

    xo_h, idx = _lowres_attn(qkv_d[0], qkv_d[1], qkv_d[2])
    xo = xo_h.transpose(1, 0, 2).reshape(HP, WP, DIM)

    qkv_win = qkv_full.reshape(HP, R, WP, R, 3, HEADS, HD)
    qkv_win = qkv_win.transpose(4, 5, 0, 2, 1, 3, 6).reshape(3, HEADS, NF, HD)

    yw_heads = []
    for hh in range(HEADS):
        yw_heads.append(_win_attn_head(idx[hh].reshape(N * TOPK),
                                       qkv_win[0, hh], qkv_win[1, hh],
                                       qkv_win[2, hh]))
    yw = jnp.stack(yw_heads)
    yw = yw.reshape(HEADS, HP, WP, R, R, HD).transpose(1, 3, 2, 4, 0, 5)
    yw = yw.reshape(H, W, DIM)

    xu = jnp.repeat(jnp.repeat(xo, R, axis=0), R, axis=1)
    xu_p = jnp.pad(xu, ((1, 1), (1, 1), (0, 0)))
    w9x = dwx_w[:, 0, :, :].reshape(DIM, 9).T
    z = _dwmix(xu_p, yw, w9x, dwx_b)

    z_p = jnp.pad(z, ((1, 1), (1, 1), (0, 0)))
    w9 = dw_w[:, 0, :, :].reshape(DIM, 9).T
    z2 = _dw(z_p, w9, dw_b)
    out = _matmul_bias(z2.reshape(NF, DIM), pw_w[:, :, 0, 0].T, pw_b, bm=BM)
    return out.reshape(1, H, W, DIM)


# speedup vs baseline: 5.9513x; 1.2216x over previous
import jax
import jax.numpy as jnp
from jax.experimental import pallas as pl
from jax.experimental.pallas import tpu as pltpu

DIM = 384
HEADS = 8
R = 4
TOPK = 4
HD = DIM // HEADS
SCALE = HD ** -0.5
H = 224
W = 224
HP = H // R
WP = W // R
N = HP * WP
NF = H * W
RR = R * R
BQ = 112
BM = 448
BW = 8
NEG = -1e30
HI = jax.lax.Precision.HIGHEST


def _matmul_kernel(x_ref, w_ref, o_ref):
    o_ref[...] = jnp.dot(x_ref[...], w_ref[...],
                         preferred_element_type=jnp.float32)


def _matmul(x, w, bm):
    m, k = x.shape
    n = w.shape[1]
    return pl.pallas_call(
        _matmul_kernel,
        grid=(m // bm,),
        in_specs=[pl.BlockSpec((bm, k), lambda i: (i, 0)),
                  pl.BlockSpec((k, n), lambda i: (0, 0))],
        out_specs=pl.BlockSpec((bm, n), lambda i: (i, 0)),
        out_shape=jax.ShapeDtypeStruct((m, n), jnp.float32),
    )(x, w)


def _matmul_t_kernel(x_ref, w_ref, o_ref):
    v = jnp.dot(x_ref[...], w_ref[...], preferred_element_type=jnp.float32)
    o_ref[...] = v.reshape(BM, 3 * HEADS, HD).transpose(1, 0, 2)


def _matmul_t(x, w):
    m, k = x.shape
    n = w.shape[1]
    return pl.pallas_call(
        _matmul_t_kernel,
        grid=(m // BM,),
        in_specs=[pl.BlockSpec((BM, k), lambda i: (i, 0)),
                  pl.BlockSpec((k, n), lambda i: (0, 0))],
        out_specs=pl.BlockSpec((3 * HEADS, BM, HD), lambda i: (0, i, 0)),
        out_shape=jax.ShapeDtypeStruct((3 * HEADS, m, HD), jnp.float32),
    )(x, w)


def _matmul_bias_kernel(x_ref, w_ref, b_ref, o_ref):
    o_ref[...] = jnp.dot(x_ref[...], w_ref[...],
                         preferred_element_type=jnp.float32) + b_ref[...]


def _matmul_bias(x, w, b, bm):
    m, k = x.shape
    n = w.shape[1]
    return pl.pallas_call(
        _matmul_bias_kernel,
        grid=(m // bm,),
        in_specs=[pl.BlockSpec((bm, k), lambda i: (i, 0)),
                  pl.BlockSpec((k, n), lambda i: (0, 0)),
                  pl.BlockSpec((1, n), lambda i: (0, 0))],
        out_specs=pl.BlockSpec((bm, n), lambda i: (i, 0)),
        out_shape=jax.ShapeDtypeStruct((m, n), jnp.float32),
    )(x, w, b.reshape(1, n))


def _pool_kernel(x_ref, o_ref):
    v = x_ref[...]
    row = jax.lax.broadcasted_iota(jnp.int32, (WP, RR * WP), 0)
    col = jax.lax.broadcasted_iota(jnp.int32, (WP, RR * WP), 1) // RR
    p = jnp.where(row == col, 1.0, 0.0).astype(jnp.float32)
    o_ref[...] = (jnp.dot(p, v, precision=HI, preferred_element_type=jnp.float32)
                  * (1.0 / RR))


def _pool(x_win):
    return pl.pallas_call(
        _pool_kernel,
        grid=(HP,),
        in_specs=[pl.BlockSpec((RR * WP, DIM), lambda i: (i, 0))],
        out_specs=pl.BlockSpec((WP, DIM), lambda i: (i, 0)),
        out_shape=jax.ShapeDtypeStruct((N, DIM), jnp.float32),
    )(x_win)


def _lowres_attn_kernel(q_ref, k_ref, v_ref, o_ref, idx_ref):
    q = q_ref[0]
    k = k_ref[0]
    v = v_ref[0]
    s = jax.lax.dot_general(q, k, (((1,), (1,)), ((), ())),
                            preferred_element_type=jnp.float32) * SCALE
    m = s.max(axis=1, keepdims=True)
    e = jnp.exp(s - m)
    d = e.sum(axis=1, keepdims=True)
    o_ref[0] = jax.lax.dot_general(e, v, (((1,), (0,)), ((), ())),
                                   preferred_element_type=jnp.float32) / d
    col = jax.lax.broadcasted_iota(jnp.int32, s.shape, 1)
    t = s
    for i in range(TOPK):
        mi = t.max(axis=1, keepdims=True)
        cand = jnp.where(t >= mi, col, jnp.int32(N))
        ji = cand.min(axis=1)
        idx_ref[0, :, i] = ji
        t = jnp.where(col == ji[:, None], NEG, t)


def _lowres_attn(q, k, v):
    return pl.pallas_call(
        _lowres_attn_kernel,
        grid=(HEADS, N // BQ),
        in_specs=[pl.BlockSpec((1, BQ, HD), lambda h, i: (h, i, 0)),
                  pl.BlockSpec((1, N, HD), lambda h, i: (h, 0, 0)),
                  pl.BlockSpec((1, N, HD), lambda h, i: (h, 0, 0))],
        out_specs=[pl.BlockSpec((1, BQ, HD), lambda h, i: (h, i, 0)),
                   pl.BlockSpec((1, BQ, TOPK), lambda h, i: (h, i, 0))],
        out_shape=[jax.ShapeDtypeStruct((HEADS, N, HD), jnp.float32),
                   jax.ShapeDtypeStruct((HEADS, N, TOPK), jnp.int32)],
    )(q, k, v)


def _win_attn_kernel(idx_ref, q_ref, k_ref, v_ref, acc_ref, o_ref):
    del acc_ref
    nb = pl.program_id(0)
    q = q_ref[0]
    kparts = []
    vparts = []
    base = nb * (BW * TOPK)
    for wloc in range(BW):
        for t in range(TOPK):
            j = idx_ref[base + wloc * TOPK + t]
            kparts.append(k_ref[0, pl.ds(j * RR, RR), :])
            vparts.append(v_ref[0, pl.ds(j * RR, RR), :])
    kc = jnp.concatenate(kparts, axis=0)
    vc = jnp.concatenate(vparts, axis=0)
    s = jax.lax.dot_general(q, kc, (((1,), (1,)), ((), ())),
                            preferred_element_type=jnp.float32) * SCALE
    rw = jax.lax.broadcasted_iota(jnp.int32, s.shape, 0) // RR
    cw = jax.lax.broadcasted_iota(jnp.int32, s.shape, 1) // (TOPK * RR)
    s = jnp.where(rw == cw, s, NEG)
    m = s.max(axis=1, keepdims=True)
    e = jnp.exp(s - m)
    d = e.sum(axis=1, keepdims=True)
    o = jax.lax.dot_general(e, vc, (((1,), (0,)), ((), ())),
                            preferred_element_type=jnp.float32) / d
    o_ref[...] = o[None]


def _win_attn_head(h, idx_flat_h, qkv_full, acc):
    return pl.pallas_call(
        _win_attn_kernel,
        grid_spec=pltpu.PrefetchScalarGridSpec(
            num_scalar_prefetch=1,
            grid=(N // BW,),
            in_specs=[pl.BlockSpec((1, BW * RR, HD), lambda i, s_: (h, i, 0)),
                      pl.BlockSpec((1, NF, HD), lambda i, s_: (HEADS + h, 0, 0)),
                      pl.BlockSpec((1, NF, HD), lambda i, s_: (2 * HEADS + h, 0, 0)),
                      pl.BlockSpec(memory_space=pl.ANY)],
            out_specs=pl.BlockSpec((1, BW * RR, HD), lambda i, s_: (h, i, 0)),
        ),
        out_shape=jax.ShapeDtypeStruct((HEADS, NF, HD), jnp.float32),
        input_output_aliases={4: 0},
        compiler_params=pltpu.CompilerParams(vmem_limit_bytes=62 * 1024 * 1024),
    )(idx_flat_h, qkv_full, qkv_full, qkv_full, acc)


def _dwmix_kernel(x0_ref, x1_ref, x2_ref, y_ref, w_ref, b_ref, o_ref):
    acc = jnp.broadcast_to(b_ref[0][None, None, :], (1, W, DIM))
    xs = (x0_ref, x1_ref, x2_ref)
    for dr in range(3):
        xv = xs[dr][...]
        for dc in range(3):
            acc = acc + xv[:, dc:dc + W, :] * w_ref[3 * dr + dc][None, None, :]
    o_ref[...] = 0.5 * acc + 0.5 * y_ref[...]


def _dwmix(xpad, yw, w9, b):
    return pl.pallas_call(
        _dwmix_kernel,
        grid=(H,),
        in_specs=[pl.BlockSpec((1, W + 2, DIM), lambda i: (i, 0, 0)),
                  pl.BlockSpec((1, W + 2, DIM), lambda i: (i + 1, 0, 0)),
                  pl.BlockSpec((1, W + 2, DIM), lambda i: (i + 2, 0, 0)),
                  pl.BlockSpec((1, W, DIM), lambda i: (i, 0, 0)),
                  pl.BlockSpec((9, DIM), lambda i: (0, 0)),
                  pl.BlockSpec((1, DIM), lambda i: (0, 0))],
        out_specs=pl.BlockSpec((1, W, DIM), lambda i: (i, 0, 0)),
        out_shape=jax.ShapeDtypeStruct((H, W, DIM), jnp.float32),
    )(xpad, xpad, xpad, yw, w9, b.reshape(1, DIM))


def _dw_kernel(x0_ref, x1_ref, x2_ref, w_ref, b_ref, o_ref):
    acc = jnp.broadcast_to(b_ref[0][None, None, :], (1, W, DIM))
    xs = (x0_ref, x1_ref, x2_ref)
    for dr in range(3):
        xv = xs[dr][...]
        for dc in range(3):
            acc = acc + xv[:, dc:dc + W, :] * w_ref[3 * dr + dc][None, None, :]
    o_ref[...] = acc


def _dw(xpad, w9, b):
    return pl.pallas_call(
        _dw_kernel,
        grid=(H,),
        in_specs=[pl.BlockSpec((1, W + 2, DIM), lambda i: (i, 0, 0)),
                  pl.BlockSpec((1, W + 2, DIM), lambda i: (i + 1, 0, 0)),
                  pl.BlockSpec((1, W + 2, DIM), lambda i: (i + 2, 0, 0)),
                  pl.BlockSpec((9, DIM), lambda i: (0, 0)),
                  pl.BlockSpec((1, DIM), lambda i: (0, 0))],
        out_specs=pl.BlockSpec((1, W, DIM), lambda i: (i, 0, 0)),
        out_shape=jax.ShapeDtypeStruct((H, W, DIM), jnp.float32),
    )(xpad, xpad, xpad, w9, b.reshape(1, DIM))


def kernel(x, qkv_w, dwx_w, dwx_b, dw_w, dw_b, pw_w, pw_b):
    wt = qkv_w.T
    x_win = x.reshape(HP, R, WP, R, DIM).transpose(0, 2, 1, 3, 4).reshape(NF, DIM)

    qkv_full = _matmul_t(x_win, wt)

    xf = _pool(x_win)
    qkv_d = _matmul(xf, wt, bm=BM)
    qkv_d = qkv_d.reshape(N, 3, HEADS, HD).transpose(1, 2, 0, 3)

    xo_h, idx = _lowres_attn(qkv_d[0], qkv_d[1], qkv_d[2])
    xo = xo_h.transpose(1, 0, 2).reshape(HP, WP, DIM)

    yw = jnp.zeros((HEADS, NF, HD), jnp.float32)
    for hh in range(HEADS):
        yw = _win_attn_head(hh, idx[hh].reshape(N * TOPK), qkv_full, yw)
    yw = yw.reshape(HEADS, HP, WP, R, R, HD).transpose(1, 3, 2, 4, 0, 5)
    yw = yw.reshape(H, W, DIM)

    xu = jnp.repeat(jnp.repeat(xo, R, axis=0), R, axis=1)
    xu_p = jnp.pad(xu, ((1, 1), (1, 1), (0, 0)))
    w9x = dwx_w[:, 0, :, :].reshape(DIM, 9).T
    z = _dwmix(xu_p, yw, w9x, dwx_b)

    z_p = jnp.pad(z, ((1, 1), (1, 1), (0, 0)))
    w9 = dw_w[:, 0, :, :].reshape(DIM, 9).T
    z2 = _dw(z_p, w9, dw_b)
    out = _matmul_bias(z2.reshape(NF, DIM), pw_w[:, :, 0, 0].T, pw_b, bm=BM)
    return out.reshape(1, H, W, DIM)


# speedup vs baseline: 6.9579x; 1.1691x over previous
import jax
import jax.numpy as jnp
from jax.experimental import pallas as pl
from jax.experimental.pallas import tpu as pltpu

DIM = 384
HEADS = 8
R = 4
TOPK = 4
HD = DIM // HEADS
SCALE = HD ** -0.5
H = 224
W = 224
HP = H // R
WP = W // R
N = HP * WP
NF = H * W
RR = R * R
BQ = 112
BM = 448
BW = 16
NEG = -1e30
HI = jax.lax.Precision.HIGHEST


def _matmul_kernel(x_ref, w_ref, o_ref):
    o_ref[...] = jnp.dot(x_ref[...], w_ref[...],
                         preferred_element_type=jnp.float32)


def _matmul(x, w, bm):
    m, k = x.shape
    n = w.shape[1]
    return pl.pallas_call(
        _matmul_kernel,
        grid=(m // bm,),
        in_specs=[pl.BlockSpec((bm, k), lambda i: (i, 0)),
                  pl.BlockSpec((k, n), lambda i: (0, 0))],
        out_specs=pl.BlockSpec((bm, n), lambda i: (i, 0)),
        out_shape=jax.ShapeDtypeStruct((m, n), jnp.float32),
    )(x, w)


def _matmul_t_kernel(x_ref, w_ref, o_ref):
    v = jnp.dot(x_ref[...], w_ref[...], preferred_element_type=jnp.float32)
    o_ref[...] = v.reshape(BM, 3 * HEADS, HD).transpose(1, 0, 2)


def _matmul_t(x, w):
    m, k = x.shape
    n = w.shape[1]
    return pl.pallas_call(
        _matmul_t_kernel,
        grid=(m // BM,),
        in_specs=[pl.BlockSpec((BM, k), lambda i: (i, 0)),
                  pl.BlockSpec((k, n), lambda i: (0, 0))],
        out_specs=pl.BlockSpec((3 * HEADS, BM, HD), lambda i: (0, i, 0)),
        out_shape=jax.ShapeDtypeStruct((3 * HEADS, m, HD), jnp.float32),
    )(x, w)


def _matmul_bias_kernel(x_ref, w_ref, b_ref, o_ref):
    o_ref[...] = jnp.dot(x_ref[...], w_ref[...],
                         preferred_element_type=jnp.float32) + b_ref[...]


def _matmul_bias(x, w, b, bm):
    m, k = x.shape
    n = w.shape[1]
    return pl.pallas_call(
        _matmul_bias_kernel,
        grid=(m // bm,),
        in_specs=[pl.BlockSpec((bm, k), lambda i: (i, 0)),
                  pl.BlockSpec((k, n), lambda i: (0, 0)),
                  pl.BlockSpec((1, n), lambda i: (0, 0))],
        out_specs=pl.BlockSpec((bm, n), lambda i: (i, 0)),
        out_shape=jax.ShapeDtypeStruct((m, n), jnp.float32),
    )(x, w, b.reshape(1, n))


def _pool_kernel(x_ref, o_ref):
    v = x_ref[...]
    row = jax.lax.broadcasted_iota(jnp.int32, (WP, RR * WP), 0)
    col = jax.lax.broadcasted_iota(jnp.int32, (WP, RR * WP), 1) // RR
    p = jnp.where(row == col, 1.0, 0.0).astype(jnp.float32)
    o_ref[...] = (jnp.dot(p, v, precision=HI, preferred_element_type=jnp.float32)
                  * (1.0 / RR))


def _pool(x_win):
    return pl.pallas_call(
        _pool_kernel,
        grid=(HP,),
        in_specs=[pl.BlockSpec((RR * WP, DIM), lambda i: (i, 0))],
        out_specs=pl.BlockSpec((WP, DIM), lambda i: (i, 0)),
        out_shape=jax.ShapeDtypeStruct((N, DIM), jnp.float32),
    )(x_win)


def _lowres_attn_kernel(q_ref, k_ref, v_ref, o_ref, idx_ref):
    q = q_ref[0]
    k = k_ref[0]
    v = v_ref[0]
    s = jax.lax.dot_general(q, k, (((1,), (1,)), ((), ())),
                            preferred_element_type=jnp.float32) * SCALE
    m = s.max(axis=1, keepdims=True)
    e = jnp.exp(s - m)
    d = e.sum(axis=1, keepdims=True)
    o_ref[0] = jax.lax.dot_general(e, v, (((1,), (0,)), ((), ())),
                                   preferred_element_type=jnp.float32) / d
    col = jax.lax.broadcasted_iota(jnp.int32, s.shape, 1)
    t = s
    for i in range(TOPK):
        mi = t.max(axis=1, keepdims=True)
        cand = jnp.where(t >= mi, col, jnp.int32(N))
        ji = cand.min(axis=1)
        idx_ref[0, :, i] = ji
        t = jnp.where(col == ji[:, None], NEG, t)


def _lowres_attn(q, k, v):
    return pl.pallas_call(
        _lowres_attn_kernel,
        grid=(HEADS, N // BQ),
        in_specs=[pl.BlockSpec((1, BQ, HD), lambda h, i: (h, i, 0)),
                  pl.BlockSpec((1, N, HD), lambda h, i: (h, 0, 0)),
                  pl.BlockSpec((1, N, HD), lambda h, i: (h, 0, 0))],
        out_specs=[pl.BlockSpec((1, BQ, HD), lambda h, i: (h, i, 0)),
                   pl.BlockSpec((1, BQ, TOPK), lambda h, i: (h, i, 0))],
        out_shape=[jax.ShapeDtypeStruct((HEADS, N, HD), jnp.float32),
                   jax.ShapeDtypeStruct((HEADS, N, TOPK), jnp.int32)],
    )(q, k, v)


def _win_attn_kernel(idx_ref, q_ref, k_ref, v_ref, acc_ref, o_ref):
    del acc_ref
    nb = pl.program_id(0)
    q = q_ref[0]
    kparts = []
    vparts = []
    base = nb * (BW * TOPK)
    for wloc in range(BW):
        for t in range(TOPK):
            j = idx_ref[base + wloc * TOPK + t]
            kparts.append(k_ref[0, pl.ds(j * RR, RR), :])
            vparts.append(v_ref[0, pl.ds(j * RR, RR), :])
    kc = jnp.concatenate(kparts, axis=0)
    vc = jnp.concatenate(vparts, axis=0)
    s = jax.lax.dot_general(q, kc, (((1,), (1,)), ((), ())),
                            preferred_element_type=jnp.float32) * SCALE
    rw = jax.lax.broadcasted_iota(jnp.int32, s.shape, 0) // RR
    cw = jax.lax.broadcasted_iota(jnp.int32, s.shape, 1) // (TOPK * RR)
    s = jnp.where(rw == cw, s, NEG)
    m = s.max(axis=1, keepdims=True)
    e = jnp.exp(s - m)
    d = e.sum(axis=1, keepdims=True)
    o = jax.lax.dot_general(e, vc, (((1,), (0,)), ((), ())),
                            preferred_element_type=jnp.float32) / d
    o_ref[...] = o[None]


def _win_attn_head(h, idx_flat_h, qkv_full, acc):
    return pl.pallas_call(
        _win_attn_kernel,
        grid_spec=pltpu.PrefetchScalarGridSpec(
            num_scalar_prefetch=1,
            grid=(N // BW,),
            in_specs=[pl.BlockSpec((1, BW * RR, HD), lambda i, s_: (h, i, 0)),
                      pl.BlockSpec((1, NF, HD), lambda i, s_: (HEADS + h, 0, 0)),
                      pl.BlockSpec((1, NF, HD), lambda i, s_: (2 * HEADS + h, 0, 0)),
                      pl.BlockSpec(memory_space=pl.ANY)],
            out_specs=pl.BlockSpec((1, BW * RR, HD), lambda i, s_: (h, i, 0)),
        ),
        out_shape=jax.ShapeDtypeStruct((HEADS, NF, HD), jnp.float32),
        input_output_aliases={4: 0},
        compiler_params=pltpu.CompilerParams(vmem_limit_bytes=62 * 1024 * 1024),
    )(idx_flat_h, qkv_full, qkv_full, qkv_full, acc)


def _dwmix_kernel(x0_ref, x1_ref, x2_ref, y_ref, w_ref, b_ref, o_ref):
    acc = jnp.broadcast_to(b_ref[0][None, None, :], (1, W, DIM))
    xs = (x0_ref, x1_ref, x2_ref)
    for dr in range(3):
        xv = xs[dr][...]
        for dc in range(3):
            acc = acc + xv[:, dc:dc + W, :] * w_ref[3 * dr + dc][None, None, :]
    o_ref[...] = 0.5 * acc + 0.5 * y_ref[...]


def _dwmix(xpad, yw, w9, b):
    return pl.pallas_call(
        _dwmix_kernel,
        grid=(H,),
        in_specs=[pl.BlockSpec((1, W + 2, DIM), lambda i: (i, 0, 0)),
                  pl.BlockSpec((1, W + 2, DIM), lambda i: (i + 1, 0, 0)),
                  pl.BlockSpec((1, W + 2, DIM), lambda i: (i + 2, 0, 0)),
                  pl.BlockSpec((1, W, DIM), lambda i: (i, 0, 0)),
                  pl.BlockSpec((9, DIM), lambda i: (0, 0)),
                  pl.BlockSpec((1, DIM), lambda i: (0, 0))],
        out_specs=pl.BlockSpec((1, W, DIM), lambda i: (i, 0, 0)),
        out_shape=jax.ShapeDtypeStruct((H, W, DIM), jnp.float32),
    )(xpad, xpad, xpad, yw, w9, b.reshape(1, DIM))


def _dw_kernel(x0_ref, x1_ref, x2_ref, w_ref, b_ref, o_ref):
    acc = jnp.broadcast_to(b_ref[0][None, None, :], (1, W, DIM))
    xs = (x0_ref, x1_ref, x2_ref)
    for dr in range(3):
        xv = xs[dr][...]
        for dc in range(3):
            acc = acc + xv[:, dc:dc + W, :] * w_ref[3 * dr + dc][None, None, :]
    o_ref[...] = acc


def _dw(xpad, w9, b):
    return pl.pallas_call(
        _dw_kernel,
        grid=(H,),
        in_specs=[pl.BlockSpec((1, W + 2, DIM), lambda i: (i, 0, 0)),
                  pl.BlockSpec((1, W + 2, DIM), lambda i: (i + 1, 0, 0)),
                  pl.BlockSpec((1, W + 2, DIM), lambda i: (i + 2, 0, 0)),
                  pl.BlockSpec((9, DIM), lambda i: (0, 0)),
                  pl.BlockSpec((1, DIM), lambda i: (0, 0))],
        out_specs=pl.BlockSpec((1, W, DIM), lambda i: (i, 0, 0)),
        out_shape=jax.ShapeDtypeStruct((H, W, DIM), jnp.float32),
    )(xpad, xpad, xpad, w9, b.reshape(1, DIM))


def kernel(x, qkv_w, dwx_w, dwx_b, dw_w, dw_b, pw_w, pw_b):
    wt = qkv_w.T
    x_win = x.reshape(HP, R, WP, R, DIM).transpose(0, 2, 1, 3, 4).reshape(NF, DIM)

    qkv_full = _matmul_t(x_win, wt)

    xf = _pool(x_win)
    qkv_d = _matmul(xf, wt, bm=BM)
    qkv_d = qkv_d.reshape(N, 3, HEADS, HD).transpose(1, 2, 0, 3)

    xo_h, idx = _lowres_attn(qkv_d[0], qkv_d[1], qkv_d[2])
    xo = xo_h.transpose(1, 0, 2).reshape(HP, WP, DIM)

    yw = jnp.zeros((HEADS, NF, HD), jnp.float32)
    for hh in range(HEADS):
        yw = _win_attn_head(hh, idx[hh].reshape(N * TOPK), qkv_full, yw)
    yw = yw.reshape(HEADS, HP, WP, R, R, HD).transpose(1, 3, 2, 4, 0, 5)
    yw = yw.reshape(H, W, DIM)

    xu = jnp.repeat(jnp.repeat(xo, R, axis=0), R, axis=1)
    xu_p = jnp.pad(xu, ((1, 1), (1, 1), (0, 0)))
    w9x = dwx_w[:, 0, :, :].reshape(DIM, 9).T
    z = _dwmix(xu_p, yw, w9x, dwx_b)

    z_p = jnp.pad(z, ((1, 1), (1, 1), (0, 0)))
    w9 = dw_w[:, 0, :, :].reshape(DIM, 9).T
    z2 = _dw(z_p, w9, dw_b)
    out = _matmul_bias(z2.reshape(NF, DIM), pw_w[:, :, 0, 0].T, pw_b, bm=BM)
    return out.reshape(1, H, W, DIM)
